# baseline (speedup 1.0000x reference)
import functools

import jax
import jax.numpy as jnp
from jax import lax
from jax.experimental import pallas as pl
from jax.experimental.pallas import tpu as pltpu
from jax.experimental.pallas import tpu_sc as plsc

VOCAB = 100000
EMBD = 32
STORY = 50
SENT = 20
HOPS = 3
BS = 1024

NW = 32
SEGS = STORY * BS
SEG_W = SEGS // NW
CH = 50
NCH = SEG_W // CH
QSEG_W = BS // NW


def _pe_weights():
    J, d = SENT, EMBD
    return [
        [1.0 - j / J - (k + 1.0 / d) * (1.0 - 2.0 * j / J) for j in range(J)]
        for k in range(HOPS)
    ]


_PE = _pe_weights()

_PERM = [v for i in range(16) for v in (i, 16 + i)]


def _bags_body(xi, qi, a0, a1, a2, a3,
               u0, w0, s1, w1, s2, w2, s3,
               idx0, idx1, rows0, rows1, acs0, acs1, acw0, acw1, qidxb,
               sg0, sg1, so0, so1):
    idxb = (idx0, idx1)
    rows = (rows0, rows1)
    accs = (acs0, acs1)
    accw = (acw0, acw1)
    sg = (sg0, sg1)
    so = (so0, so1)
    wid = lax.axis_index("s") * 2 + lax.axis_index("c")

    def seg_sum(rbuf, abuf_s, abuf_w, wts):
        def seg(i, _):
            b = i * SENT
            s_lo = jnp.zeros((16,), jnp.float32)
            s_hi = jnp.zeros((16,), jnp.float32)
            v_lo = jnp.zeros((16,), jnp.float32)
            v_hi = jnp.zeros((16,), jnp.float32)
            for j in range(SENT):
                v = rbuf[b + j, :]
                r_lo = plsc.bitcast(
                    lax.shift_left(v, jnp.int32(16)), jnp.float32)
                r_hi = plsc.bitcast(
                    lax.bitwise_and(v, jnp.int32(-65536)), jnp.float32)
                if abuf_s is not None:
                    s_lo = s_lo + r_lo
                    s_hi = s_hi + r_hi
                if abuf_w is not None:
                    v_lo = v_lo + r_lo * wts[j]
                    v_hi = v_hi + r_hi * wts[j]
            if abuf_s is not None:
                abuf_s[i, pl.ds(0, 16)] = s_lo
                abuf_s[i, pl.ds(16, 16)] = s_hi
            if abuf_w is not None:
                abuf_w[i, pl.ds(0, 16)] = v_lo
                abuf_w[i, pl.ds(16, 16)] = v_hi
            return 0

        return seg

    def x_pass(table, out_s, out_w, wts):
        def gather_start(t, b):
            seg0 = wid * SEG_W + t * CH
            pltpu.sync_copy(xi.at[pl.ds(seg0 * SENT, CH * SENT)], idxb[b])
            pltpu.async_copy(table.at[idxb[b]], rows[b], sg[b])

        def gather_wait(b):
            pltpu.make_async_copy(table.at[idxb[b]], rows[b], sg[b]).wait()

        def flush_start(t, b):
            seg0 = wid * SEG_W + t * CH
            if out_s is not None:
                pltpu.async_copy(accs[b], out_s.at[pl.ds(seg0, CH)], so[b])
            if out_w is not None:
                pltpu.async_copy(accw[b], out_w.at[pl.ds(seg0, CH)], so[b])

        def flush_wait(b):
            if out_s is not None:
                pltpu.make_async_copy(
                    accs[b], out_s.at[pl.ds(0, CH)], so[b]).wait()
            if out_w is not None:
                pltpu.make_async_copy(
                    accw[b], out_w.at[pl.ds(0, CH)], so[b]).wait()

        gather_start(0, 0)

        def two(tt, _):
            for b in (0, 1):
                t = tt * 2 + b

                @pl.when(t + 1 < NCH)
                def _():
                    gather_start(t + 1, 1 - b)

                gather_wait(b)

                @pl.when(t >= 2)
                def _():
                    flush_wait(b)

                lax.fori_loop(0, CH, seg_sum(rows[b], accs[b] if out_s is not None else None,
                                             accw[b] if out_w is not None else None, wts), 0)
                flush_start(t, b)
            return 0

        lax.fori_loop(0, NCH // 2, two, 0)
        flush_wait(0)
        flush_wait(1)

    qseg0 = wid * QSEG_W
    pltpu.sync_copy(qi.at[pl.ds(qseg0 * SENT, QSEG_W * SENT)], qidxb)
    pltpu.async_copy(a0.at[qidxb], rows0.at[pl.ds(0, QSEG_W * SENT)], sg0).wait()
    lax.fori_loop(0, QSEG_W,
                  seg_sum(rows0, acs0, None, None), 0)
    pltpu.sync_copy(acs0.at[pl.ds(0, QSEG_W)], u0.at[pl.ds(qseg0, QSEG_W)])

    x_pass(a0, None, w0, _PE[0])
    x_pass(a1, s1, w1, _PE[1])
    x_pass(a2, s2, w2, _PE[2])
    x_pass(a3, s3, None, None)


@functools.cache
def _make_bags():
  return pl.kernel(
    _bags_body,
    mesh=plsc.VectorSubcoreMesh(core_axis_name="c", subcore_axis_name="s"),
    out_type=[
        jax.ShapeDtypeStruct((BS, EMBD), jnp.float32),
        jax.ShapeDtypeStruct((SEGS, EMBD), jnp.float32),
        jax.ShapeDtypeStruct((SEGS, EMBD), jnp.float32),
        jax.ShapeDtypeStruct((SEGS, EMBD), jnp.float32),
        jax.ShapeDtypeStruct((SEGS, EMBD), jnp.float32),
        jax.ShapeDtypeStruct((SEGS, EMBD), jnp.float32),
        jax.ShapeDtypeStruct((SEGS, EMBD), jnp.float32),
    ],
    scratch_types=[
        pltpu.VMEM((CH * SENT,), jnp.int32),
        pltpu.VMEM((CH * SENT,), jnp.int32),
        pltpu.VMEM((CH * SENT, EMBD // 2), jnp.int32),
        pltpu.VMEM((CH * SENT, EMBD // 2), jnp.int32),
        pltpu.VMEM((CH, EMBD), jnp.float32),
        pltpu.VMEM((CH, EMBD), jnp.float32),
        pltpu.VMEM((CH, EMBD), jnp.float32),
        pltpu.VMEM((CH, EMBD), jnp.float32),
        pltpu.VMEM((QSEG_W * SENT,), jnp.int32),
        pltpu.SemaphoreType.DMA,
        pltpu.SemaphoreType.DMA,
        pltpu.SemaphoreType.DMA,
        pltpu.SemaphoreType.DMA,
    ],
    compiler_params=pltpu.CompilerParams(
        use_tc_tiling_on_sc=False, needs_layout_passes=False),
  )


def _hops_body(u0, w0, s1, w1, s2, w2, s3, ta, tc, u3):
    ta_b = ta[...][:, :, None]
    tc_b = tc[...][:, :, None]
    u = u0[...]
    for m_ref, c_ref in ((w0, s1), (w1, s2), (w2, s3)):
        m = m_ref[...] + ta_b
        c = c_ref[...] + tc_b
        p = jnp.sum(m * u[None, :, :], axis=2)
        p = p - jnp.max(p, axis=0, keepdims=True)
        e = jnp.exp(p)
        p = e / jnp.sum(e, axis=0, keepdims=True)
        o = jnp.sum(c * p[:, :, None], axis=0)
        u = u + o
    u3[...] = u


def _proj_body(u3, a3, out, mmax, ssum):
    p = pl.program_id(0)
    v = pl.program_id(1)
    logits = lax.dot_general(
        u3[...].astype(jnp.bfloat16), a3[...].astype(jnp.bfloat16),
        (((1,), (1,)), ((), ())),
        preferred_element_type=jnp.float32)
    col = lax.broadcasted_iota(jnp.int32, logits.shape, 1) + v * _VT
    logits = jnp.where(col < VOCAB, logits, -jnp.inf)

    @pl.when(p == 0)
    def _():
        tmax = jnp.max(logits, axis=1, keepdims=True)

        @pl.when(v == 0)
        def _():
            mmax[...] = tmax
            ssum[...] = jnp.sum(jnp.exp(logits - tmax), axis=1, keepdims=True)

        @pl.when(v > 0)
        def _():
            nm = jnp.maximum(mmax[...], tmax)
            ssum[...] = ssum[...] * jnp.exp(mmax[...] - nm) + jnp.sum(
                jnp.exp(logits - nm), axis=1, keepdims=True)
            mmax[...] = nm

    @pl.when(p == 1)
    def _():
        out[...] = logits - mmax[...] - jnp.log(ssum[...])


_VT = 2048
_NV = -(-VOCAB // _VT)
_BT = 64


def kernel(x, q, A0, A1, A2, A3, TA, TC):
    xi = jnp.transpose(x, (1, 0, 2)).reshape(-1)
    qi = q.reshape(-1)
    ta = TA.reshape(STORY, 1)
    tc = TC.reshape(STORY, 1)

    perm = jnp.asarray(_PERM, dtype=jnp.int32)
    a0b, a1b, a2b, a3b = (
        lax.bitcast_convert_type(
            a[:, perm].astype(jnp.bfloat16).reshape(VOCAB, EMBD // 2, 2),
            jnp.int32)
        for a in (A0, A1, A2, A3))
    u0, w0, s1, w1, s2, w2, s3 = _make_bags()(xi, qi, a0b, a1b, a2b, a3b)

    bag3 = lambda a: a.reshape(STORY, BS, EMBD)
    u3 = pl.pallas_call(
        _hops_body,
        grid=(BS // _BT,),
        in_specs=[
            pl.BlockSpec((_BT, EMBD), lambda b: (b, 0)),
            pl.BlockSpec((STORY, _BT, EMBD), lambda b: (0, b, 0)),
            pl.BlockSpec((STORY, _BT, EMBD), lambda b: (0, b, 0)),
            pl.BlockSpec((STORY, _BT, EMBD), lambda b: (0, b, 0)),
            pl.BlockSpec((STORY, _BT, EMBD), lambda b: (0, b, 0)),
            pl.BlockSpec((STORY, _BT, EMBD), lambda b: (0, b, 0)),
            pl.BlockSpec((STORY, _BT, EMBD), lambda b: (0, b, 0)),
            pl.BlockSpec((STORY, 1), lambda b: (0, 0)),
            pl.BlockSpec((STORY, 1), lambda b: (0, 0)),
        ],
        out_specs=pl.BlockSpec((_BT, EMBD), lambda b: (b, 0)),
        out_shape=jax.ShapeDtypeStruct((BS, EMBD), jnp.float32),
    )(u0, bag3(w0), bag3(s1), bag3(w1), bag3(s2), bag3(w2), bag3(s3), ta, tc)

    out = pl.pallas_call(
        _proj_body,
        grid=(2, _NV),
        in_specs=[
            pl.BlockSpec((BS, EMBD), lambda p, v: (0, 0)),
            pl.BlockSpec((_VT, EMBD), lambda p, v: (v, 0)),
        ],
        out_specs=pl.BlockSpec((BS, _VT), lambda p, v: (0, v * p)),
        out_shape=jax.ShapeDtypeStruct((BS, VOCAB), jnp.float32),
        scratch_shapes=[
            pltpu.VMEM((BS, 1), jnp.float32),
            pltpu.VMEM((BS, 1), jnp.float32),
        ],
        compiler_params=pltpu.CompilerParams(
            dimension_semantics=("arbitrary", "arbitrary")),
    )(u3, A3)
    return out


# speedup vs baseline: 1.3503x; 1.3503x over previous
import functools

import jax
import jax.numpy as jnp
from jax import lax
from jax.experimental import pallas as pl
from jax.experimental.pallas import tpu as pltpu
from jax.experimental.pallas import tpu_sc as plsc

VOCAB = 100000
EMBD = 32
STORY = 50
SENT = 20
HOPS = 3
BS = 1024

NW = 32
SEGS = STORY * BS
SEG_W = SEGS // NW
CH = 50
NCH = SEG_W // CH
QSEG_W = BS // NW


def _pe_weights():
    J, d = SENT, EMBD
    return [
        [1.0 - j / J - (k + 1.0 / d) * (1.0 - 2.0 * j / J) for j in range(J)]
        for k in range(HOPS)
    ]


_PE = _pe_weights()


def _bags_body(xi, qi, a0, a1, a2, a3,
               u0, w0, s1, w1, s2, w2, s3,
               idx0, idx1, rows0, rows1, acs0, acs1, acw0, acw1, qidxb,
               sg0, sg1, so0, so1):
    idxb = (idx0, idx1)
    rows = (rows0, rows1)
    accs = (acs0, acs1)
    accw = (acw0, acw1)
    sg = (sg0, sg1)
    so = (so0, so1)
    wid = lax.axis_index("s") * 2 + lax.axis_index("c")

    def seg_sum(rbuf, abuf_s, abuf_w, wts):
        def seg(i, _):
            b = i * SENT
            s_lo = jnp.zeros((16,), jnp.float32)
            s_hi = jnp.zeros((16,), jnp.float32)
            v_lo = jnp.zeros((16,), jnp.float32)
            v_hi = jnp.zeros((16,), jnp.float32)
            for j in range(SENT):
                r_lo = rbuf[b + j, pl.ds(0, 16)]
                r_hi = rbuf[b + j, pl.ds(16, 16)]
                if abuf_s is not None:
                    s_lo = s_lo + r_lo
                    s_hi = s_hi + r_hi
                if abuf_w is not None:
                    v_lo = v_lo + r_lo * wts[j]
                    v_hi = v_hi + r_hi * wts[j]
            if abuf_s is not None:
                abuf_s[i, pl.ds(0, 16)] = s_lo
                abuf_s[i, pl.ds(16, 16)] = s_hi
            if abuf_w is not None:
                abuf_w[i, pl.ds(0, 16)] = v_lo
                abuf_w[i, pl.ds(16, 16)] = v_hi
            return 0

        return seg

    def x_pass(table, out_s, out_w, wts):
        def gather_start(t, b):
            seg0 = wid * SEG_W + t * CH
            pltpu.sync_copy(xi.at[pl.ds(seg0 * SENT, CH * SENT)], idxb[b])
            pltpu.async_copy(table.at[idxb[b]], rows[b], sg[b])

        def gather_wait(b):
            pltpu.make_async_copy(table.at[idxb[b]], rows[b], sg[b]).wait()

        def flush_start(t, b):
            seg0 = wid * SEG_W + t * CH
            if out_s is not None:
                pltpu.async_copy(accs[b], out_s.at[pl.ds(seg0, CH)], so[b])
            if out_w is not None:
                pltpu.async_copy(accw[b], out_w.at[pl.ds(seg0, CH)], so[b])

        def flush_wait(b):
            if out_s is not None:
                pltpu.make_async_copy(
                    accs[b], out_s.at[pl.ds(0, CH)], so[b]).wait()
            if out_w is not None:
                pltpu.make_async_copy(
                    accw[b], out_w.at[pl.ds(0, CH)], so[b]).wait()

        gather_start(0, 0)

        def two(tt, _):
            for b in (0, 1):
                t = tt * 2 + b

                @pl.when(t + 1 < NCH)
                def _():
                    gather_start(t + 1, 1 - b)

                gather_wait(b)

                @pl.when(t >= 2)
                def _():
                    flush_wait(b)

                lax.fori_loop(0, CH, seg_sum(rows[b], accs[b] if out_s is not None else None,
                                             accw[b] if out_w is not None else None, wts), 0)
                flush_start(t, b)
            return 0

        lax.fori_loop(0, NCH // 2, two, 0)
        flush_wait(0)
        flush_wait(1)

    qseg0 = wid * QSEG_W
    pltpu.sync_copy(qi.at[pl.ds(qseg0 * SENT, QSEG_W * SENT)], qidxb)
    pltpu.async_copy(a0.at[qidxb], rows0.at[pl.ds(0, QSEG_W * SENT)], sg0).wait()
    lax.fori_loop(0, QSEG_W,
                  seg_sum(rows0, acs0, None, None), 0)
    pltpu.sync_copy(acs0.at[pl.ds(0, QSEG_W)], u0.at[pl.ds(qseg0, QSEG_W)])

    x_pass(a0, None, w0, _PE[0])
    x_pass(a1, s1, w1, _PE[1])
    x_pass(a2, s2, w2, _PE[2])
    x_pass(a3, s3, None, None)


@functools.cache
def _make_bags():
  return pl.kernel(
    _bags_body,
    mesh=plsc.VectorSubcoreMesh(core_axis_name="c", subcore_axis_name="s"),
    out_type=[
        jax.ShapeDtypeStruct((BS, EMBD), jnp.float32),
        jax.ShapeDtypeStruct((SEGS, EMBD), jnp.float32),
        jax.ShapeDtypeStruct((SEGS, EMBD), jnp.float32),
        jax.ShapeDtypeStruct((SEGS, EMBD), jnp.float32),
        jax.ShapeDtypeStruct((SEGS, EMBD), jnp.float32),
        jax.ShapeDtypeStruct((SEGS, EMBD), jnp.float32),
        jax.ShapeDtypeStruct((SEGS, EMBD), jnp.float32),
    ],
    scratch_types=[
        pltpu.VMEM((CH * SENT,), jnp.int32),
        pltpu.VMEM((CH * SENT,), jnp.int32),
        pltpu.VMEM((CH * SENT, EMBD), jnp.float32),
        pltpu.VMEM((CH * SENT, EMBD), jnp.float32),
        pltpu.VMEM((CH, EMBD), jnp.float32),
        pltpu.VMEM((CH, EMBD), jnp.float32),
        pltpu.VMEM((CH, EMBD), jnp.float32),
        pltpu.VMEM((CH, EMBD), jnp.float32),
        pltpu.VMEM((QSEG_W * SENT,), jnp.int32),
        pltpu.SemaphoreType.DMA,
        pltpu.SemaphoreType.DMA,
        pltpu.SemaphoreType.DMA,
        pltpu.SemaphoreType.DMA,
    ],
    compiler_params=pltpu.CompilerParams(
        use_tc_tiling_on_sc=False, needs_layout_passes=False),
  )


def _hops_body(u0, w0, s1, w1, s2, w2, s3, ta, tc, u3):
    ta_b = ta[...][:, :, None]
    tc_b = tc[...][:, :, None]
    u = u0[...]
    for m_ref, c_ref in ((w0, s1), (w1, s2), (w2, s3)):
        m = m_ref[...] + ta_b
        c = c_ref[...] + tc_b
        p = jnp.sum(m * u[None, :, :], axis=2)
        p = p - jnp.max(p, axis=0, keepdims=True)
        e = jnp.exp(p)
        p = e / jnp.sum(e, axis=0, keepdims=True)
        o = jnp.sum(c * p[:, :, None], axis=0)
        u = u + o
    u3[...] = u


def _proj_body(u3, a3, out, mmax, ssum):
    p = pl.program_id(0)
    v = pl.program_id(1)
    logits = lax.dot_general(
        u3[...].astype(jnp.bfloat16), a3[...].astype(jnp.bfloat16),
        (((1,), (1,)), ((), ())),
        preferred_element_type=jnp.float32)
    col = lax.broadcasted_iota(jnp.int32, logits.shape, 1) + v * _VT
    logits = jnp.where(col < VOCAB, logits, -jnp.inf)

    @pl.when(p == 0)
    def _():
        @pl.when(v == 0)
        def _():
            mmax[...] = jnp.max(logits, axis=1, keepdims=True)
            ssum[...] = jnp.zeros_like(ssum)

        ssum[...] += jnp.sum(jnp.exp(logits - mmax[...]), axis=1,
                             keepdims=True)

    @pl.when(p == 1)
    def _():
        out[...] = logits - mmax[...] - jnp.log(ssum[...])


_VT = 2048
_NV = -(-VOCAB // _VT)
_BT = 64


def kernel(x, q, A0, A1, A2, A3, TA, TC):
    xi = jnp.transpose(x, (1, 0, 2)).reshape(-1)
    qi = q.reshape(-1)
    ta = TA.reshape(STORY, 1)
    tc = TC.reshape(STORY, 1)

    u0, w0, s1, w1, s2, w2, s3 = _make_bags()(xi, qi, A0, A1, A2, A3)

    bag3 = lambda a: a.reshape(STORY, BS, EMBD)
    u3 = pl.pallas_call(
        _hops_body,
        grid=(BS // _BT,),
        in_specs=[
            pl.BlockSpec((_BT, EMBD), lambda b: (b, 0)),
            pl.BlockSpec((STORY, _BT, EMBD), lambda b: (0, b, 0)),
            pl.BlockSpec((STORY, _BT, EMBD), lambda b: (0, b, 0)),
            pl.BlockSpec((STORY, _BT, EMBD), lambda b: (0, b, 0)),
            pl.BlockSpec((STORY, _BT, EMBD), lambda b: (0, b, 0)),
            pl.BlockSpec((STORY, _BT, EMBD), lambda b: (0, b, 0)),
            pl.BlockSpec((STORY, _BT, EMBD), lambda b: (0, b, 0)),
            pl.BlockSpec((STORY, 1), lambda b: (0, 0)),
            pl.BlockSpec((STORY, 1), lambda b: (0, 0)),
        ],
        out_specs=pl.BlockSpec((_BT, EMBD), lambda b: (b, 0)),
        out_shape=jax.ShapeDtypeStruct((BS, EMBD), jnp.float32),
    )(u0, bag3(w0), bag3(s1), bag3(w1), bag3(s2), bag3(w2), bag3(s3), ta, tc)

    out = pl.pallas_call(
        _proj_body,
        grid=(2, _NV),
        in_specs=[
            pl.BlockSpec((BS, EMBD), lambda p, v: (0, 0)),
            pl.BlockSpec((_VT, EMBD), lambda p, v: (v, 0)),
        ],
        out_specs=pl.BlockSpec((BS, _VT), lambda p, v: (0, v * p)),
        out_shape=jax.ShapeDtypeStruct((BS, VOCAB), jnp.float32),
        scratch_shapes=[
            pltpu.VMEM((BS, 1), jnp.float32),
            pltpu.VMEM((BS, 1), jnp.float32),
        ],
        compiler_params=pltpu.CompilerParams(
            dimension_semantics=("arbitrary", "arbitrary")),
    )(u3, A3)
    return out


# speedup vs baseline: 1.4289x; 1.0582x over previous
import functools

import jax
import jax.numpy as jnp
from jax import lax
from jax.experimental import pallas as pl
from jax.experimental.pallas import tpu as pltpu
from jax.experimental.pallas import tpu_sc as plsc

VOCAB = 100000
EMBD = 32
STORY = 50
SENT = 20
HOPS = 3
BS = 1024

NW = 32
SEGS = STORY * BS
SEG_W = SEGS // NW
CH = 50
NCH = SEG_W // CH
QSEG_W = BS // NW


def _pe_weights():
    J, d = SENT, EMBD
    return [
        [1.0 - j / J - (k + 1.0 / d) * (1.0 - 2.0 * j / J) for j in range(J)]
        for k in range(HOPS)
    ]


_PE = _pe_weights()


def _bags_body(spec, with_q, *refs):
    n_t = len(spec)
    n_out = (1 if with_q else 0) + sum(int(ws) + int(ww) for ws, ww, _ in spec)
    pos = 0
    xi = refs[pos]; pos += 1
    qi = None
    if with_q:
        qi = refs[pos]; pos += 1
    tabs = refs[pos:pos + n_t]; pos += n_t
    outs = list(refs[pos:pos + n_out]); pos += n_out
    (idx0, idx1, rows0, rows1, acs0, acs1, acw0, acw1, qidxb,
     sg0, sg1, so0, so1) = refs[pos:pos + 13]
    idxb = (idx0, idx1)
    rows = (rows0, rows1)
    accs = (acs0, acs1)
    accw = (acw0, acw1)
    sg = (sg0, sg1)
    so = (so0, so1)
    wid = lax.axis_index("s") * 2 + lax.axis_index("c")

    def seg_sum(rbuf, abuf_s, abuf_w, wts):
        def seg(i, _):
            b = i * SENT
            s_lo = jnp.zeros((16,), jnp.float32)
            s_hi = jnp.zeros((16,), jnp.float32)
            v_lo = jnp.zeros((16,), jnp.float32)
            v_hi = jnp.zeros((16,), jnp.float32)
            for j in range(SENT):
                r_lo = rbuf[b + j, pl.ds(0, 16)]
                r_hi = rbuf[b + j, pl.ds(16, 16)]
                if abuf_s is not None:
                    s_lo = s_lo + r_lo
                    s_hi = s_hi + r_hi
                if abuf_w is not None:
                    v_lo = v_lo + r_lo * wts[j]
                    v_hi = v_hi + r_hi * wts[j]
            if abuf_s is not None:
                abuf_s[i, pl.ds(0, 16)] = s_lo
                abuf_s[i, pl.ds(16, 16)] = s_hi
            if abuf_w is not None:
                abuf_w[i, pl.ds(0, 16)] = v_lo
                abuf_w[i, pl.ds(16, 16)] = v_hi
            return 0

        return seg

    def x_pass(table, out_s, out_w, wts):
        def gather_start(t, b):
            seg0 = wid * SEG_W + t * CH
            pltpu.sync_copy(xi.at[pl.ds(seg0 * SENT, CH * SENT)], idxb[b])
            pltpu.async_copy(table.at[idxb[b]], rows[b], sg[b])

        def gather_wait(b):
            pltpu.make_async_copy(table.at[idxb[b]], rows[b], sg[b]).wait()

        def flush_start(t, b):
            seg0 = wid * SEG_W + t * CH
            if out_s is not None:
                pltpu.async_copy(accs[b], out_s.at[pl.ds(seg0, CH)], so[b])
            if out_w is not None:
                pltpu.async_copy(accw[b], out_w.at[pl.ds(seg0, CH)], so[b])

        def flush_wait(b):
            if out_s is not None:
                pltpu.make_async_copy(
                    accs[b], out_s.at[pl.ds(0, CH)], so[b]).wait()
            if out_w is not None:
                pltpu.make_async_copy(
                    accw[b], out_w.at[pl.ds(0, CH)], so[b]).wait()

        gather_start(0, 0)

        def two(tt, _):
            for b in (0, 1):
                t = tt * 2 + b

                @pl.when(t + 1 < NCH)
                def _():
                    gather_start(t + 1, 1 - b)

                gather_wait(b)

                @pl.when(t >= 2)
                def _():
                    flush_wait(b)

                lax.fori_loop(0, CH, seg_sum(rows[b], accs[b] if out_s is not None else None,
                                             accw[b] if out_w is not None else None, wts), 0)
                flush_start(t, b)
            return 0

        lax.fori_loop(0, NCH // 2, two, 0)
        flush_wait(0)
        flush_wait(1)

    oi = 0
    if with_q:
        u0 = outs[0]
        oi = 1
        qseg0 = wid * QSEG_W
        pltpu.sync_copy(qi.at[pl.ds(qseg0 * SENT, QSEG_W * SENT)], qidxb)
        pltpu.async_copy(
            tabs[0].at[qidxb], rows0.at[pl.ds(0, QSEG_W * SENT)], sg0).wait()
        lax.fori_loop(0, QSEG_W, seg_sum(rows0, acs0, None, None), 0)
        pltpu.sync_copy(acs0.at[pl.ds(0, QSEG_W)], u0.at[pl.ds(qseg0, QSEG_W)])

    for (ws, ww, k), table in zip(spec, tabs):
        out_s = out_w = None
        if ws:
            out_s = outs[oi]
            oi += 1
        if ww:
            out_w = outs[oi]
            oi += 1
        x_pass(table, out_s, out_w, _PE[k] if k is not None else None)


@functools.cache
def _make_bags(spec, with_q):
  n_out = (1 if with_q else 0) + sum(int(ws) + int(ww) for ws, ww, _ in spec)
  return pl.kernel(
    functools.partial(_bags_body, spec, with_q),
    mesh=plsc.VectorSubcoreMesh(core_axis_name="c", subcore_axis_name="s"),
    out_type=(
        ([jax.ShapeDtypeStruct((BS, EMBD), jnp.float32)] if with_q else [])
        + [jax.ShapeDtypeStruct((SEGS, EMBD), jnp.float32)]
        * (n_out - (1 if with_q else 0))
    ),
    scratch_types=[
        pltpu.VMEM((CH * SENT,), jnp.int32),
        pltpu.VMEM((CH * SENT,), jnp.int32),
        pltpu.VMEM((CH * SENT, EMBD), jnp.float32),
        pltpu.VMEM((CH * SENT, EMBD), jnp.float32),
        pltpu.VMEM((CH, EMBD), jnp.float32),
        pltpu.VMEM((CH, EMBD), jnp.float32),
        pltpu.VMEM((CH, EMBD), jnp.float32),
        pltpu.VMEM((CH, EMBD), jnp.float32),
        pltpu.VMEM((QSEG_W * SENT,), jnp.int32),
        pltpu.SemaphoreType.DMA,
        pltpu.SemaphoreType.DMA,
        pltpu.SemaphoreType.DMA,
        pltpu.SemaphoreType.DMA,
    ],
    compiler_params=pltpu.CompilerParams(
        use_tc_tiling_on_sc=False, needs_layout_passes=False),
  )


def _hops_body(u0, w0, s1, w1, s2, w2, s3, ta, tc, u3):
    ta_b = ta[...][:, :, None]
    tc_b = tc[...][:, :, None]
    u = u0[...]
    for m_ref, c_ref in ((w0, s1), (w1, s2), (w2, s3)):
        m = m_ref[...] + ta_b
        c = c_ref[...] + tc_b
        p = jnp.sum(m * u[None, :, :], axis=2)
        p = p - jnp.max(p, axis=0, keepdims=True)
        e = jnp.exp(p)
        p = e / jnp.sum(e, axis=0, keepdims=True)
        o = jnp.sum(c * p[:, :, None], axis=0)
        u = u + o
    u3[...] = u


def _proj_body(u3, a3, out, mmax, ssum):
    p = pl.program_id(0)
    v = pl.program_id(1)
    logits = lax.dot_general(
        u3[...].astype(jnp.bfloat16), a3[...].astype(jnp.bfloat16),
        (((1,), (1,)), ((), ())),
        preferred_element_type=jnp.float32)
    col = lax.broadcasted_iota(jnp.int32, logits.shape, 1) + v * _VT
    logits = jnp.where(col < VOCAB, logits, -jnp.inf)

    @pl.when(p == 0)
    def _():
        @pl.when(v == 0)
        def _():
            mmax[...] = jnp.max(logits, axis=1, keepdims=True)
            ssum[...] = jnp.zeros_like(ssum)

        ssum[...] += jnp.sum(jnp.exp(logits - mmax[...]), axis=1,
                             keepdims=True)

    @pl.when(p == 1)
    def _():
        out[...] = logits - mmax[...] - jnp.log(ssum[...])


_VT = 2048
_NV = -(-VOCAB // _VT)
_BT = 64


def kernel(x, q, A0, A1, A2, A3, TA, TC):
    xi = jnp.transpose(x, (1, 0, 2)).reshape(-1)
    qi = q.reshape(-1)
    ta = TA.reshape(STORY, 1)
    tc = TC.reshape(STORY, 1)

    s1, w1, s2, w2 = _make_bags(
        ((True, True, 1), (True, True, 2)), False)(xi, A1, A2)
    u0, w0, s3 = _make_bags(
        ((False, True, 0), (True, False, None)), True)(xi, qi, A0, A3)

    bag3 = lambda a: a.reshape(STORY, BS, EMBD)
    u3 = pl.pallas_call(
        _hops_body,
        grid=(BS // _BT,),
        in_specs=[
            pl.BlockSpec((_BT, EMBD), lambda b: (b, 0)),
            pl.BlockSpec((STORY, _BT, EMBD), lambda b: (0, b, 0)),
            pl.BlockSpec((STORY, _BT, EMBD), lambda b: (0, b, 0)),
            pl.BlockSpec((STORY, _BT, EMBD), lambda b: (0, b, 0)),
            pl.BlockSpec((STORY, _BT, EMBD), lambda b: (0, b, 0)),
            pl.BlockSpec((STORY, _BT, EMBD), lambda b: (0, b, 0)),
            pl.BlockSpec((STORY, _BT, EMBD), lambda b: (0, b, 0)),
            pl.BlockSpec((STORY, 1), lambda b: (0, 0)),
            pl.BlockSpec((STORY, 1), lambda b: (0, 0)),
        ],
        out_specs=pl.BlockSpec((_BT, EMBD), lambda b: (b, 0)),
        out_shape=jax.ShapeDtypeStruct((BS, EMBD), jnp.float32),
    )(u0, bag3(w0), bag3(s1), bag3(w1), bag3(s2), bag3(w2), bag3(s3), ta, tc)

    out = pl.pallas_call(
        _proj_body,
        grid=(2, _NV),
        in_specs=[
            pl.BlockSpec((BS, EMBD), lambda p, v: (0, 0)),
            pl.BlockSpec((_VT, EMBD), lambda p, v: (v, 0)),
        ],
        out_specs=pl.BlockSpec((BS, _VT), lambda p, v: (0, v * p)),
        out_shape=jax.ShapeDtypeStruct((BS, VOCAB), jnp.float32),
        scratch_shapes=[
            pltpu.VMEM((BS, 1), jnp.float32),
            pltpu.VMEM((BS, 1), jnp.float32),
        ],
        compiler_params=pltpu.CompilerParams(
            dimension_semantics=("arbitrary", "arbitrary")),
    )(u3, A3)
    return out
